# Initial kernel scaffold; baseline (speedup 1.0000x reference)
import jax
import jax.numpy as jnp
from jax.experimental import pallas as pl


def kernel(pos_u, pos_v, neg_v, u_table, v_table):
    raise NotImplementedError("write your pallas kernel here")


# speedup vs baseline: 1.5941x; 1.5941x over previous
import functools

import jax
import jax.numpy as jnp
from jax import lax
from jax.experimental import pallas as pl
from jax.experimental.pallas import tpu as pltpu
from jax.experimental.pallas import tpu_sc as plsc

D = 64
B = 16384
NEG = 5
NC = 2
NS = 16
NW = NC * NS
PER_W = B // NW
CH = 128
NCHUNK = PER_W // CH
NSLOT = 2 + NEG


def _softplus(x):
    y = 1.0 + jnp.exp(x)
    bits = lax.bitcast_convert_type(y, jnp.int32)
    e = lax.shift_right_logical(bits, 23) - 127
    m = lax.bitcast_convert_type(
        (bits & 0x007FFFFF) | 0x3F800000, jnp.float32)
    big = m > 1.4142135
    m = jnp.where(big, m * 0.5, m)
    ef = (e + big.astype(jnp.int32)).astype(jnp.float32)
    t = m - 1.0
    z = t * t
    p = jnp.float32(7.0376836292e-2)
    for coef in (-1.1514610310e-1, 1.1676998740e-1, -1.2420140846e-1,
                 1.4249322787e-1, -1.6668057665e-1, 2.0000714765e-1,
                 -2.4999993993e-1, 3.3333331174e-1):
        p = p * t + coef
    r = t * z * p - 0.5 * z + t
    return ef * 0.69314718 + r


def _body(u_hbm, v_hbm, idx_hbm, out_hbm,
          idx_v, r0, r1, r2, r3, r4, r5, r6, acc_v, sem):
    wid = lax.axis_index("s") * NC + lax.axis_index("c")
    pltpu.sync_copy(idx_hbm.at[wid], idx_v)
    acc_v[...] = jnp.zeros((16,), jnp.float32)
    rows = [r0, r1, r2, r3, r4, r5, r6]
    iota16 = lax.iota(jnp.int32, 16)

    def chunk_body(c, carry):
        cops = []
        for j in range(NSLOT):
            tbl = u_hbm if j == 0 else v_hbm
            cops.append(pltpu.async_copy(tbl.at[idx_v.at[c, j]], rows[j], sem))
        for cp in cops:
            cp.wait()

        def group_body(g, carry2):
            r16 = g * 16 + iota16
            pacc = jnp.zeros((16,), jnp.float32)
            nacc = [jnp.zeros((16,), jnp.float32) for _ in range(NEG)]
            for d in range(D):
                dsplat = jnp.full((16,), d, jnp.int32)
                u_d = plsc.load_gather(r0, [r16, dsplat])
                v_d = plsc.load_gather(r1, [r16, dsplat])
                pacc = pacc + u_d * v_d
                for k in range(NEG):
                    nk_d = plsc.load_gather(rows[2 + k], [r16, dsplat])
                    nacc[k] = nacc[k] + u_d * nk_d
            loss = _softplus(-jnp.clip(pacc, -10.0, 10.0))
            for k in range(NEG):
                loss = loss + _softplus(jnp.clip(nacc[k], -10.0, 10.0))
            acc_v[...] = acc_v[...] + loss
            return 0

        lax.fori_loop(0, CH // 16, group_body, 0)
        return 0

    lax.fori_loop(0, NCHUNK, chunk_body, 0)
    pltpu.sync_copy(acc_v, out_hbm.at[wid])


@jax.jit
def kernel(pos_u, pos_v, neg_v, u_table, v_table):
    pos_u = pos_u.astype(jnp.int32)
    pos_v = pos_v.astype(jnp.int32)
    neg_v = neg_v.astype(jnp.int32)
    pu = pos_u.reshape(NW, NCHUNK, 1, CH)
    pv = pos_v.reshape(NW, NCHUNK, 1, CH)
    ng = neg_v.reshape(NW, NCHUNK, CH, NEG).transpose(0, 1, 3, 2)
    idx_all = jnp.concatenate([pu, pv, ng], axis=2)

    mesh = plsc.VectorSubcoreMesh(core_axis_name="c", subcore_axis_name="s")
    run = pl.kernel(
        _body,
        out_type=jax.ShapeDtypeStruct((NW, 16), jnp.float32),
        mesh=mesh,
        compiler_params=pltpu.CompilerParams(
            needs_layout_passes=False, use_tc_tiling_on_sc=False),
        scratch_types=[
            pltpu.VMEM((NCHUNK, NSLOT, CH), jnp.int32),
            *[pltpu.VMEM((CH, D), jnp.float32) for _ in range(NSLOT)],
            pltpu.VMEM((16,), jnp.float32),
            pltpu.SemaphoreType.DMA,
        ],
    )
    partials = run(u_table, v_table, idx_all)
    return jnp.sum(partials) / B


# speedup vs baseline: 3.8405x; 2.4092x over previous
import jax
import jax.numpy as jnp
from jax import lax
from jax.experimental import pallas as pl
from jax.experimental.pallas import tpu as pltpu
from jax.experimental.pallas import tpu_sc as plsc

D = 64
B = 16384
NEG = 5
NC = 2
NS = 16
NW = NC * NS
PER_W = B // NW
CH = 32
NCHUNK = PER_W // CH
NSLOT = 2 + NEG
NSTREAM = 2 * NSLOT
IDX_W = NCHUNK * NSTREAM * CH
CONV_C = 8192

VOCAB = 1000000
GRID = (VOCAB + 2 * CONV_C - 1) // (2 * CONV_C)
OUT_ROWS = GRID * CONV_C
TAIL_S = VOCAB - (GRID - 1) * 2 * CONV_C
B0 = (GRID - 1) * CONV_C + TAIL_S
DUMMY_N = OUT_ROWS - B0


def _softplus(x):
    y = 1.0 + jnp.exp(x)
    bits = lax.bitcast_convert_type(y, jnp.int32)
    e = lax.shift_right_logical(bits, 23) - 127
    m = lax.bitcast_convert_type(
        (bits & 0x007FFFFF) | 0x3F800000, jnp.float32)
    big = m > 1.4142135
    m = jnp.where(big, m * 0.5, m)
    ef = (e + big.astype(jnp.int32)).astype(jnp.float32)
    t = m - 1.0
    z = t * t
    p = jnp.float32(7.0376836292e-2)
    for coef in (-1.1514610310e-1, 1.1676998740e-1, -1.2420140846e-1,
                 1.4249322787e-1, -1.6668057665e-1, 2.0000714765e-1,
                 -2.4999993993e-1, 3.3333331174e-1):
        p = p * t + coef
    r = t * z * p - 0.5 * z + t
    return ef * 0.69314718 + r


def _body(u_hbm, v_hbm, idx_hbm, out_hbm,
          idx_v, rows, stage_v, acc_v, sems):
    wid = lax.axis_index("s") * NC + lax.axis_index("c")
    pltpu.sync_copy(idx_hbm.at[pl.ds(wid * IDX_W, IDX_W)], idx_v)
    zeros = jnp.zeros((16,), jnp.float32)
    for i in range(8):
        acc_v[pl.ds(i * 16, 16)] = zeros
    iota16 = lax.iota(jnp.int32, 16)
    lane0 = iota16 == 0

    def issue(c, bf):
        for t in range(NSTREAM):
            tbl = u_hbm if t < 2 else v_hbm
            pltpu.async_copy(
                tbl.at[idx_v.at[pl.ds((c * NSTREAM + t) * CH, CH)]],
                rows.at[bf, t], sems.at[bf])

    issue(0, 0)

    def chunk_body(c, carry):
        bf = lax.rem(c, 2)

        @pl.when(c + 1 < NCHUNK)
        def _():
            issue(c + 1, lax.rem(c + 1, 2))

        for t in range(NSTREAM):
            tbl = u_hbm if t < 2 else v_hbm
            pltpu.make_async_copy(
                tbl.at[idx_v.at[pl.ds((c * NSTREAM + t) * CH, CH)]],
                rows.at[bf, t], sems.at[bf]).wait()

        def elem_body(e, carry2):
            u = [rows[bf, 0, e, pl.ds(16 * k, 16)] +
                 rows[bf, 1, e, pl.ds(D + 16 * k, 16)] for k in range(4)]
            for j in range(1, NSLOT):
                w = [rows[bf, 2 * j, e, pl.ds(16 * k, 16)] +
                     rows[bf, 2 * j + 1, e, pl.ds(D + 16 * k, 16)]
                     for k in range(4)]
                part = u[0] * w[0] + u[1] * w[1] + u[2] * w[2] + u[3] * w[3]
                sc = jnp.sum(part)
                plsc.store_scatter(stage_v.at[pl.ds((j - 1) * CH, CH)],
                                   [jnp.full((16,), 0, jnp.int32) + e],
                                   zeros + sc, mask=lane0)
            return 0

        lax.fori_loop(0, CH, elem_body, 0)

        def group_body(g, carry2):
            pos = stage_v[pl.ds(g * 16, 16)]
            loss = _softplus(-jnp.clip(pos, -10.0, 10.0))
            for k in range(NEG):
                nk = stage_v[pl.ds((1 + k) * CH + g * 16, 16)]
                loss = loss + _softplus(jnp.clip(nk, -10.0, 10.0))
            acc_v[pl.ds(0, 16)] = acc_v[pl.ds(0, 16)] + loss
            return 0

        lax.fori_loop(0, CH // 16, group_body, 0)
        return 0

    lax.fori_loop(0, NCHUNK, chunk_body, 0)
    pltpu.sync_copy(acc_v, out_hbm.at[pl.ds(wid * 128, 128)])


def _conv_body(ut_ref, vt_ref, u2_ref, v2_ref):
    j = pl.program_id(0)
    t = ut_ref[...].T
    s = vt_ref[...].T
    rows_lo = j * CONV_C + lax.broadcasted_iota(jnp.int32, (CONV_C, 2 * D), 0)
    valid = rows_lo < B0
    u2_ref[:, 0:D] = jnp.where(valid[:, 0:D], t[0:CONV_C], 0.0)
    u2_ref[:, D:2 * D] = jnp.where(valid[:, D:2 * D],
                                   t[CONV_C:2 * CONV_C], 0.0)
    v2_ref[:, 0:D] = jnp.where(valid[:, 0:D], s[0:CONV_C], 0.0)
    v2_ref[:, D:2 * D] = jnp.where(valid[:, D:2 * D],
                                   s[CONV_C:2 * CONV_C], 0.0)


@jax.jit
def kernel(pos_u, pos_v, neg_v, u_table, v_table):
    pos_u = pos_u.astype(jnp.int32)
    pos_v = pos_v.astype(jnp.int32)
    neg_v = neg_v.astype(jnp.int32)
    ut = u_table.T
    vt = v_table.T
    u2, v2 = pl.pallas_call(
        _conv_body,
        grid=(GRID,),
        in_specs=[
            pl.BlockSpec((D, 2 * CONV_C), lambda j: (0, j)),
            pl.BlockSpec((D, 2 * CONV_C), lambda j: (0, j)),
        ],
        out_specs=[
            pl.BlockSpec((CONV_C, 2 * D), lambda j: (j, 0)),
            pl.BlockSpec((CONV_C, 2 * D), lambda j: (j, 0)),
        ],
        out_shape=[
            jax.ShapeDtypeStruct((OUT_ROWS, 2 * D), jnp.float32),
            jax.ShapeDtypeStruct((OUT_ROWS, 2 * D), jnp.float32),
        ],
    )(ut, vt)
    pu = pos_u.reshape(NW, NCHUNK, 1, CH)
    pv = pos_v.reshape(NW, NCHUNK, 1, CH)
    ng = neg_v.reshape(NW, NCHUNK, CH, NEG).transpose(0, 1, 3, 2)
    idx7 = jnp.concatenate([pu, pv, ng], axis=2)
    half = (idx7 // (2 * CONV_C)) * CONV_C + (idx7 % CONV_C)
    par = (idx7 // CONV_C) % 2
    pos = jnp.arange(NW * NCHUNK * NSLOT * CH,
                     dtype=jnp.int32).reshape(idx7.shape)
    dummy = B0 + pos % DUMMY_N
    le = jnp.where(par == 0, half, dummy)
    lo = jnp.where(par == 1, half, dummy)
    idx_all = jnp.stack([le, lo], axis=3).reshape(-1)

    mesh = plsc.VectorSubcoreMesh(core_axis_name="c", subcore_axis_name="s")
    run = pl.kernel(
        _body,
        out_type=jax.ShapeDtypeStruct((NW * 128,), jnp.float32),
        mesh=mesh,
        compiler_params=pltpu.CompilerParams(
            needs_layout_passes=False, use_tc_tiling_on_sc=True),
        scratch_types=[
            pltpu.VMEM((IDX_W,), jnp.int32),
            pltpu.VMEM((2, NSTREAM, CH, 2 * D), jnp.float32),
            pltpu.VMEM(((1 + NEG) * CH,), jnp.float32),
            pltpu.VMEM((128,), jnp.float32),
            pltpu.SemaphoreType.DMA((2,)),
        ],
    )
    partials = run(u2, v2, idx_all)
    return jnp.sum(partials) / B
